# baseline (speedup 1.0000x reference)
import jax
import jax.numpy as jnp
from jax.experimental import pallas as pl
from jax.experimental.pallas import tpu as pltpu

_N, _K, _D = 10000, 16, 256
_B = 1000
_K2 = _K // 2


def _gat_block(self_ref, neigh_a_ref, neigh_b_ref, lens_ref, w_ref, a2_ref,
               out_ref):
    x = self_ref[...]
    nb_a = neigh_a_ref[...]
    nb_b = neigh_b_ref[...]
    w = w_ref[...]
    a2 = a2_ref[...]
    lens = lens_ref[...]

    wa = jnp.dot(w, a2, preferred_element_type=jnp.float32)
    wa_s = wa[:, 0]
    wa_n = wa[:, 1]

    nb3_a = nb_a.reshape(_B, _K2, _D)
    nb3_b = nb_b.reshape(_B, _K2, _D)

    e_self = jnp.sum(x * wa_s[None, :], axis=1, keepdims=True)
    en_a = jnp.sum(nb3_a * wa_n[None, None, :], axis=2)
    en_b = jnp.sum(nb3_b * wa_n[None, None, :], axis=2)
    e = e_self + jnp.concatenate([en_a, en_b], axis=1)
    e = jnp.where(e > 0, e, 0.2 * e)

    valid = jax.lax.broadcasted_iota(jnp.int32, (_B, _K), 1) < jnp.maximum(lens, 1)
    e = jnp.where(valid, e, -1e9)

    m = jnp.max(e, axis=1, keepdims=True)
    p = jnp.exp(e - m)
    alpha = p / jnp.sum(p, axis=1, keepdims=True)

    xagg = (x + jnp.sum(alpha[:, :_K2, None] * nb3_a, axis=1)
              + jnp.sum(alpha[:, _K2:, None] * nb3_b, axis=1))
    z = jnp.dot(xagg, w, preferred_element_type=jnp.float32)
    out_ref[...] = jnp.where(z > 0, z, jnp.exp(jnp.minimum(z, 0.0)) - 1.0)


def kernel(self_nodes, neigh_nodes, len_adj_nodes, W, a_self, a_neigh):
    neigh2 = neigh_nodes.reshape(_N, _K * _D)
    lens2 = len_adj_nodes.astype(jnp.int32).reshape(_N, 1)
    a2 = jnp.stack([a_self, a_neigh], axis=1)

    grid = (_N // _B,)
    half = _K2 * _D
    return pl.pallas_call(
        _gat_block,
        grid=grid,
        in_specs=[
            pl.BlockSpec((_B, _D), lambda i: (i, 0)),
            pl.BlockSpec((_B, half), lambda i: (i, 0)),
            pl.BlockSpec((_B, half), lambda i: (i, 1)),
            pl.BlockSpec((_B, 1), lambda i: (i, 0)),
            pl.BlockSpec((_D, _D), lambda i: (0, 0)),
            pl.BlockSpec((_D, 2), lambda i: (0, 0)),
        ],
        out_specs=pl.BlockSpec((_B, _D), lambda i: (i, 0)),
        out_shape=jax.ShapeDtypeStruct((_N, _D), jnp.float32),
        compiler_params=pltpu.CompilerParams(
            dimension_semantics=("parallel",),
        ),
    )(self_nodes, neigh2, neigh2, lens2, W, a2)


# speedup vs baseline: 2.5891x; 2.5891x over previous
import jax
import jax.numpy as jnp
from jax.experimental import pallas as pl
from jax.experimental.pallas import tpu as pltpu

_N, _K, _D = 10000, 16, 256
_B = 400
_H = _B // 2


def _gat_half(x, nb3, lens, w, wa_s, wa_n):
    e_self = jnp.sum(x * wa_s[None, :], axis=1, keepdims=True)
    e_neigh = jnp.sum(nb3 * wa_n[None, None, :], axis=2)
    e = e_self + e_neigh
    e = jnp.where(e > 0, e, 0.2 * e)

    valid = jax.lax.broadcasted_iota(jnp.int32, (_H, _K), 1) < jnp.maximum(lens, 1)
    e = jnp.where(valid, e, -1e9)

    m = jnp.max(e, axis=1, keepdims=True)
    p = jnp.exp(e - m)
    alpha = p / jnp.sum(p, axis=1, keepdims=True)

    xagg = x + jnp.sum(alpha[:, :, None] * nb3, axis=1)
    z = jnp.dot(xagg, w, preferred_element_type=jnp.float32)
    return jnp.where(z > 0, z, jnp.exp(jnp.minimum(z, 0.0)) - 1.0)


def _gat_block(self_ref, neigh_a_ref, neigh_b_ref, lens_ref, w_ref, a2_ref,
               out_ref):
    w = w_ref[...]
    a2 = a2_ref[...]
    wa = jnp.dot(w, a2, preferred_element_type=jnp.float32)
    wa_s = wa[:, 0]
    wa_n = wa[:, 1]

    out_ref[:_H, :] = _gat_half(
        self_ref[:_H, :], neigh_a_ref[...].reshape(_H, _K, _D),
        lens_ref[:_H, :], w, wa_s, wa_n)
    out_ref[_H:, :] = _gat_half(
        self_ref[_H:, :], neigh_b_ref[...].reshape(_H, _K, _D),
        lens_ref[_H:, :], w, wa_s, wa_n)


def kernel(self_nodes, neigh_nodes, len_adj_nodes, W, a_self, a_neigh):
    neigh2 = neigh_nodes.reshape(_N * _K, _D)
    lens2 = len_adj_nodes.astype(jnp.int32).reshape(_N, 1)
    a2 = jnp.stack([a_self, a_neigh], axis=1)

    grid = (_N // _B,)
    return pl.pallas_call(
        _gat_block,
        grid=grid,
        in_specs=[
            pl.BlockSpec((_B, _D), lambda i: (i, 0)),
            pl.BlockSpec((_H * _K, _D), lambda i: (2 * i, 0)),
            pl.BlockSpec((_H * _K, _D), lambda i: (2 * i + 1, 0)),
            pl.BlockSpec((_B, 1), lambda i: (i, 0)),
            pl.BlockSpec((_D, _D), lambda i: (0, 0)),
            pl.BlockSpec((_D, 2), lambda i: (0, 0)),
        ],
        out_specs=pl.BlockSpec((_B, _D), lambda i: (i, 0)),
        out_shape=jax.ShapeDtypeStruct((_N, _D), jnp.float32),
        compiler_params=pltpu.CompilerParams(
            dimension_semantics=("parallel",),
        ),
    )(self_nodes, neigh2, neigh2, lens2, W, a2)


# speedup vs baseline: 3.0464x; 1.1766x over previous
import jax
import jax.numpy as jnp
from jax.experimental import pallas as pl
from jax.experimental.pallas import tpu as pltpu

_N, _K, _D = 10000, 16, 256
_B = 1000


def _gat_block(self_ref, neigh_ref, lens_ref, w_ref, a2_ref, out_ref):
    x = self_ref[...]
    nb = neigh_ref[...]
    w = w_ref[...]
    a2 = a2_ref[...]
    lens = lens_ref[...]

    wa = jnp.dot(w, a2, preferred_element_type=jnp.float32)
    wa_s = wa[:, 0]
    wa_n = wa[:, 1]

    nb3 = nb.reshape(_B, _K, _D)
    e_self = jnp.sum(x * wa_s[None, :], axis=1, keepdims=True)
    e_neigh = jnp.sum(nb3 * wa_n[None, None, :], axis=2)

    e = e_self + e_neigh
    e = jnp.where(e > 0, e, 0.2 * e)

    valid = jax.lax.broadcasted_iota(jnp.int32, (_B, _K), 1) < jnp.maximum(lens, 1)
    e = jnp.where(valid, e, -1e9)

    m = jnp.max(e, axis=1, keepdims=True)
    p = jnp.exp(e - m)
    alpha = p / jnp.sum(p, axis=1, keepdims=True)

    xagg = x + jnp.sum(alpha[:, :, None] * nb3, axis=1)
    z = jnp.dot(xagg, w, preferred_element_type=jnp.float32)
    out_ref[...] = jnp.where(z > 0, z, jnp.exp(jnp.minimum(z, 0.0)) - 1.0)


def kernel(self_nodes, neigh_nodes, len_adj_nodes, W, a_self, a_neigh):
    neigh2 = neigh_nodes.reshape(_N * _K, _D)
    lens2 = len_adj_nodes.astype(jnp.int32).reshape(_N, 1)
    a2 = jnp.stack([a_self, a_neigh], axis=1)

    grid = (_N // _B,)
    return pl.pallas_call(
        _gat_block,
        grid=grid,
        in_specs=[
            pl.BlockSpec((_B, _D), lambda i: (i, 0)),
            pl.BlockSpec((_B * _K, _D), lambda i: (i, 0)),
            pl.BlockSpec((_B, 1), lambda i: (i, 0)),
            pl.BlockSpec((_D, _D), lambda i: (0, 0)),
            pl.BlockSpec((_D, 2), lambda i: (0, 0)),
        ],
        out_specs=pl.BlockSpec((_B, _D), lambda i: (i, 0)),
        out_shape=jax.ShapeDtypeStruct((_N, _D), jnp.float32),
        compiler_params=pltpu.CompilerParams(
            dimension_semantics=("parallel",),
        ),
    )(self_nodes, neigh2, lens2, W, a2)


# speedup vs baseline: 3.0473x; 1.0003x over previous
import jax
import jax.numpy as jnp
from jax.experimental import pallas as pl
from jax.experimental.pallas import tpu as pltpu

_N, _K, _D = 10000, 16, 256
_B = 1000


def _gat_block(self_ref, neigh_ref, lens_ref, w_ref, a2_ref, out_ref):
    x = self_ref[...]
    nb = neigh_ref[...]
    w = w_ref[...]
    a2 = a2_ref[...]
    lens = lens_ref[...]

    wa = jnp.dot(w, a2, preferred_element_type=jnp.float32)
    wa_s = wa[:, 0]
    wa_n = wa[:, 1]

    nb3 = nb.reshape(_B, _K, _D)
    e_self = jnp.sum(x * wa_s[None, :], axis=1, keepdims=True)
    e_neigh = jnp.sum(nb3 * wa_n[None, None, :], axis=2)

    e = e_self + e_neigh
    e = jnp.where(e > 0, e, 0.2 * e)

    valid = jax.lax.broadcasted_iota(jnp.int32, (_B, _K), 1) < jnp.maximum(lens, 1)
    e = jnp.where(valid, e, -1e9)

    m = jnp.max(e, axis=1, keepdims=True)
    p = jnp.exp(e - m)
    alpha = p / jnp.sum(p, axis=1, keepdims=True)

    xagg = x + jnp.sum(alpha[:, :, None] * nb3, axis=1)
    z = jnp.dot(xagg, w, preferred_element_type=jnp.float32)
    out_ref[...] = jnp.where(z > 0, z, jnp.exp(jnp.minimum(z, 0.0)) - 1.0)


def kernel(self_nodes, neigh_nodes, len_adj_nodes, W, a_self, a_neigh):
    neigh2 = neigh_nodes.reshape(_N * _K, _D)
    lens2 = len_adj_nodes.astype(jnp.int32).reshape(_N, 1)
    a2 = jnp.stack([a_self, a_neigh], axis=1)

    grid = (_N // _B,)
    return pl.pallas_call(
        _gat_block,
        grid=grid,
        in_specs=[
            pl.BlockSpec((_B, _D), lambda i: (i, 0)),
            pl.BlockSpec((_B * _K, _D), lambda i: (i, 0)),
            pl.BlockSpec((_B, 1), lambda i: (i, 0)),
            pl.BlockSpec((_D, _D), lambda i: (0, 0)),
            pl.BlockSpec((_D, 2), lambda i: (0, 0)),
        ],
        out_specs=pl.BlockSpec((_B, _D), lambda i: (i, 0)),
        out_shape=jax.ShapeDtypeStruct((_N, _D), jnp.float32),
        compiler_params=pltpu.CompilerParams(
            dimension_semantics=("arbitrary",),
        ),
    )(self_nodes, neigh2, lens2, W, a2)
